# baseline (speedup 1.0000x reference)
import functools

import jax
import jax.numpy as jnp
from jax import lax
from jax.experimental import pallas as pl
from jax.experimental.pallas import tpu as pltpu
from jax.experimental.pallas import tpu_sc as plsc

N_ROWS = 16384
N_COLS = 1024
NC = 2
NS = 16
NW = NC * NS
ROWS_PER_W = N_ROWS // NW
CHUNK = 16
N_CHUNKS = ROWS_PER_W // CHUNK
LANES = 16

_mesh = plsc.VectorSubcoreMesh(core_axis_name="c", subcore_axis_name="s")


@functools.partial(
    pl.kernel,
    mesh=_mesh,
    out_type=jax.ShapeDtypeStruct((N_ROWS, N_COLS), jnp.float32),
    scratch_types=[
        pltpu.VMEM((CHUNK, N_COLS), jnp.float32),
        pltpu.VMEM((CHUNK, N_COLS), jnp.float32),
        pltpu.VMEM((CHUNK, N_COLS), jnp.float32),
        pltpu.VMEM((CHUNK, N_COLS), jnp.float32),
        pltpu.VMEM((N_COLS,), jnp.float32),
        pltpu.VMEM((N_COLS,), jnp.float32),
        pltpu.SemaphoreType.DMA,
        pltpu.SemaphoreType.DMA,
        pltpu.SemaphoreType.DMA,
        pltpu.SemaphoreType.DMA,
    ],
)
def _sc_fma(x_hbm, mu_hbm, sigma_hbm, out_hbm,
            in0, in1, ot0, ot1, mu_v, sig_v,
            isem0, isem1, osem0, osem1):
    ins = (in0, in1)
    ots = (ot0, ot1)
    isems = (isem0, isem1)
    osems = (osem0, osem1)
    wid = lax.axis_index("s") * NC + lax.axis_index("c")
    base = wid * ROWS_PER_W
    pltpu.sync_copy(mu_hbm, mu_v)
    pltpu.sync_copy(sigma_hbm, sig_v)

    def step(i, carry):
        for b in range(2):
            def col_body(c, carry2):
                m = mu_v[pl.ds(c * LANES, LANES)]
                s = sig_v[pl.ds(c * LANES, LANES)]
                for r in range(CHUNK):
                    ots[b][r, pl.ds(c * LANES, LANES)] = (
                        m + ins[b][r, pl.ds(c * LANES, LANES)] * s
                    )
                return carry2

            lax.fori_loop(0, N_COLS // LANES, col_body, 0)
        return carry

    lax.fori_loop(0, N_CHUNKS // 2, step, 0)
    pltpu.sync_copy(ots[0], out_hbm.at[pl.ds(base, CHUNK)])


def kernel(x, mu, sigma):
    return _sc_fma(x, mu, sigma)


# speedup vs baseline: 1.1869x; 1.1869x over previous
import functools

import jax
import jax.numpy as jnp
from jax import lax
from jax.experimental import pallas as pl
from jax.experimental.pallas import tpu as pltpu
from jax.experimental.pallas import tpu_sc as plsc

N_ROWS = 16384
N_COLS = 1024
NC = 2
NS = 16
NW = NC * NS
ROWS_PER_W = N_ROWS // NW
CHUNK = 16
N_CHUNKS = ROWS_PER_W // CHUNK
LANES = 16

_mesh = plsc.VectorSubcoreMesh(core_axis_name="c", subcore_axis_name="s")


@functools.partial(
    pl.kernel,
    mesh=_mesh,
    out_type=jax.ShapeDtypeStruct((N_ROWS, N_COLS), jnp.float32),
    scratch_types=[
        pltpu.VMEM((CHUNK, N_COLS), jnp.float32),
        pltpu.VMEM((CHUNK, N_COLS), jnp.float32),
        pltpu.VMEM((CHUNK, N_COLS), jnp.float32),
        pltpu.VMEM((CHUNK, N_COLS), jnp.float32),
        pltpu.VMEM((N_COLS,), jnp.float32),
        pltpu.VMEM((N_COLS,), jnp.float32),
        pltpu.SemaphoreType.DMA,
        pltpu.SemaphoreType.DMA,
        pltpu.SemaphoreType.DMA,
        pltpu.SemaphoreType.DMA,
    ],
)
def _sc_fma(x_hbm, mu_hbm, sigma_hbm, out_hbm,
            in0, in1, ot0, ot1, mu_v, sig_v,
            isem0, isem1, osem0, osem1):
    ins = (in0, in1)
    ots = (ot0, ot1)
    isems = (isem0, isem1)
    osems = (osem0, osem1)
    wid = lax.axis_index("s") * NC + lax.axis_index("c")
    base = wid * ROWS_PER_W
    pltpu.sync_copy(mu_hbm, mu_v)
    pltpu.sync_copy(sigma_hbm, sig_v)

    for b in range(2):
        pltpu.async_copy(x_hbm.at[pl.ds(base + b * CHUNK, CHUNK)],
                         ins[b], isems[b])

    def step(i, carry):
        g = i * 2
        for b in range(2):
            k = g + b
            row0 = base + k * CHUNK
            pltpu.make_async_copy(x_hbm.at[pl.ds(row0, CHUNK)],
                                  ins[b], isems[b]).wait()

            @pl.when(i >= 1)
            def _():
                pltpu.make_async_copy(
                    ots[b], out_hbm.at[pl.ds(row0 - 2 * CHUNK, CHUNK)],
                    osems[b]).wait()

            @plsc.parallel_loop(0, N_COLS // LANES, unroll=4)
            def _(c):
                m = mu_v[pl.ds(c * LANES, LANES)]
                s = sig_v[pl.ds(c * LANES, LANES)]
                for r in range(CHUNK):
                    ots[b][r, pl.ds(c * LANES, LANES)] = (
                        m + ins[b][r, pl.ds(c * LANES, LANES)] * s
                    )

            pltpu.async_copy(ots[b], out_hbm.at[pl.ds(row0, CHUNK)],
                             osems[b])

            @pl.when(i <= N_CHUNKS // 2 - 2)
            def _():
                pltpu.async_copy(x_hbm.at[pl.ds(row0 + 2 * CHUNK, CHUNK)],
                                 ins[b], isems[b])
        return carry

    lax.fori_loop(0, N_CHUNKS // 2, step, 0)

    for b in range(2):
        row0 = base + (N_CHUNKS - 2 + b) * CHUNK
        pltpu.make_async_copy(ots[b], out_hbm.at[pl.ds(row0, CHUNK)],
                              osems[b]).wait()


def kernel(x, mu, sigma):
    return _sc_fma(x, mu, sigma)


# speedup vs baseline: 2.0972x; 1.7670x over previous
import jax
import jax.numpy as jnp
from jax.experimental import pallas as pl

N_ROWS = 16384
N_COLS = 1024
BM = 2048


def _fma_kernel(x_ref, mu_ref, sigma_ref, o_ref):
    o_ref[...] = mu_ref[...] + x_ref[...] * sigma_ref[...]


def kernel(x, mu, sigma):
    mu2 = mu.reshape(1, N_COLS)
    sigma2 = sigma.reshape(1, N_COLS)
    return pl.pallas_call(
        _fma_kernel,
        grid=(N_ROWS // BM,),
        in_specs=[
            pl.BlockSpec((BM, N_COLS), lambda i: (i, 0)),
            pl.BlockSpec((1, N_COLS), lambda i: (0, 0)),
            pl.BlockSpec((1, N_COLS), lambda i: (0, 0)),
        ],
        out_specs=pl.BlockSpec((BM, N_COLS), lambda i: (i, 0)),
        out_shape=jax.ShapeDtypeStruct((N_ROWS, N_COLS), x.dtype),
    )(x, mu2, sigma2)
